# Initial kernel scaffold; baseline (speedup 1.0000x reference)
import jax
import jax.numpy as jnp
from jax.experimental import pallas as pl


def kernel(input, embed):
    raise NotImplementedError("write your pallas kernel here")


# speedup vs baseline: 1.0002x; 1.0002x over previous
import functools

import jax
import jax.numpy as jnp
from jax import lax
from jax.experimental import pallas as pl
from jax.experimental.pallas import tpu as pltpu
from jax.experimental.pallas import tpu_sc as plsc

_DIM = 64
_N_EMBED = 1024
_BETA = 0.25

_TILE_M = 512


def _dist_body(x_ref, e_ref, ind_ref, loss_ref):
    pid = pl.program_id(0)
    x = x_ref[...]
    e = e_ref[...]
    s = jax.lax.dot_general(
        x, e, (((1,), (0,)), ((), ())), preferred_element_type=jnp.float32
    )
    x2 = jnp.sum(x * x, axis=1, keepdims=True)
    e2 = jnp.sum(e * e, axis=0, keepdims=True)
    dist = x2 - 2.0 * s + e2
    neg = -dist
    ind_ref[0, 0, :] = jnp.argmax(neg, axis=1).astype(jnp.int32)
    part = -jnp.sum(jnp.max(neg, axis=1))

    @pl.when(pid == 0)
    def _():
        loss_ref[0, 0] = 0.0

    loss_ref[0, 0] += part


def _dist_argmin(flatten, embed):
    rows = flatten.shape[0]
    grid = rows // _TILE_M
    ind, loss_sum = pl.pallas_call(
        _dist_body,
        grid=(grid,),
        in_specs=[
            pl.BlockSpec((_TILE_M, _DIM), lambda i: (i, 0)),
            pl.BlockSpec((_DIM, _N_EMBED), lambda i: (0, 0)),
        ],
        out_specs=[
            pl.BlockSpec((1, 1, _TILE_M), lambda i: (i, 0, 0)),
            pl.BlockSpec(memory_space=pltpu.SMEM),
        ],
        out_shape=[
            jax.ShapeDtypeStruct((grid, 1, _TILE_M), jnp.int32),
            jax.ShapeDtypeStruct((1, 1), jnp.float32),
        ],
    )(flatten, embed)
    return ind.reshape(rows), loss_sum[0, 0]


def _make_gather(rows):
    nw = 32
    b_per_w = rows // nw
    mesh = plsc.VectorSubcoreMesh(core_axis_name="c", subcore_axis_name="s")

    @functools.partial(
        pl.kernel,
        mesh=mesh,
        compiler_params=pltpu.CompilerParams(use_tc_tiling_on_sc=False),
        out_type=jax.ShapeDtypeStruct((rows, _DIM), jnp.float32),
        scratch_types=[
            pltpu.VMEM((b_per_w,), jnp.int32),
            pltpu.VMEM((b_per_w, _DIM), jnp.float32),
            pltpu.SemaphoreType.DMA,
        ],
    )
    def gather(table_hbm, idx_hbm, out_hbm, idx_v, rows_v, sem):
        wid = lax.axis_index("s") * 2 + lax.axis_index("c")
        base = wid * b_per_w
        pltpu.sync_copy(idx_hbm.at[pl.ds(base, b_per_w)], idx_v)
        pltpu.async_copy(table_hbm.at[idx_v], rows_v, sem).wait()
        pltpu.sync_copy(rows_v, out_hbm.at[pl.ds(base, b_per_w)])

    return gather


def kernel(input, embed):
    b, t, c = input.shape
    flatten = input.reshape(-1, _DIM)
    rows = flatten.shape[0]
    ind, loss_sum = _dist_argmin(flatten, embed)
    table = embed.T
    quantize = _make_gather(rows)(table, ind)
    loss = loss_sum * (_BETA / (rows * _DIM))
    return quantize.reshape(b, t, c), loss, ind.reshape(b, t)


# speedup vs baseline: 1.0516x; 1.0513x over previous
import functools

import jax
import jax.numpy as jnp
from jax import lax
from jax.experimental import pallas as pl
from jax.experimental.pallas import tpu as pltpu
from jax.experimental.pallas import tpu_sc as plsc

_DIM = 64
_N_EMBED = 1024
_BETA = 0.25

_TILE_M = 512


def _dist_body(x_ref, e_ref, ind_ref, loss_ref):
    pid = pl.program_id(0)
    x = x_ref[...]
    e = e_ref[...]
    s = jax.lax.dot_general(
        x, e, (((1,), (0,)), ((), ())), preferred_element_type=jnp.float32
    )
    x2 = jnp.sum(x * x, axis=1, keepdims=True)
    e2 = jnp.sum(e * e, axis=0, keepdims=True)
    dist = x2 - 2.0 * s + e2
    m = jnp.min(dist, axis=1, keepdims=True)
    cols = jax.lax.broadcasted_iota(jnp.int32, dist.shape, 1).astype(jnp.float32)
    hit = jnp.where(dist <= m, cols, float(_N_EMBED))
    ind_ref[0, 0, :] = jnp.min(hit, axis=1).astype(jnp.int32)
    part = jnp.sum(m)

    @pl.when(pid == 0)
    def _():
        loss_ref[0, 0] = 0.0

    loss_ref[0, 0] += part


def _dist_argmin(flatten, embed):
    rows = flatten.shape[0]
    grid = rows // _TILE_M
    ind, loss_sum = pl.pallas_call(
        _dist_body,
        grid=(grid,),
        in_specs=[
            pl.BlockSpec((_TILE_M, _DIM), lambda i: (i, 0)),
            pl.BlockSpec((_DIM, _N_EMBED), lambda i: (0, 0)),
        ],
        out_specs=[
            pl.BlockSpec((1, 1, _TILE_M), lambda i: (i, 0, 0)),
            pl.BlockSpec(memory_space=pltpu.SMEM),
        ],
        out_shape=[
            jax.ShapeDtypeStruct((grid, 1, _TILE_M), jnp.int32),
            jax.ShapeDtypeStruct((1, 1), jnp.float32),
        ],
    )(flatten, embed)
    return ind.reshape(rows), loss_sum[0, 0]


def _make_gather(rows):
    nw = 32
    b_per_w = rows // nw
    groups = b_per_w // 16
    mesh = plsc.VectorSubcoreMesh(core_axis_name="c", subcore_axis_name="s")

    @functools.partial(
        pl.kernel,
        mesh=mesh,
        compiler_params=pltpu.CompilerParams(
            use_tc_tiling_on_sc=False, needs_layout_passes=False
        ),
        out_type=jax.ShapeDtypeStruct((nw, b_per_w * _DIM), jnp.float32),
        scratch_types=[
            pltpu.VMEM((_DIM * _N_EMBED,), jnp.float32),
            pltpu.VMEM((b_per_w,), jnp.int32),
            pltpu.VMEM((b_per_w * _DIM,), jnp.float32),
            pltpu.SemaphoreType.DMA,
        ],
    )
    def gather(table_hbm, idx_hbm, out_hbm, table_v, idx_v, rows_v, sem):
        wid = lax.axis_index("s") * 2 + lax.axis_index("c")
        cp = pltpu.async_copy(table_hbm, table_v, sem)
        pltpu.sync_copy(idx_hbm.at[pl.ds(wid * b_per_w, b_per_w)], idx_v)
        cp.wait()
        lanes = lax.iota(jnp.int32, 16)

        def body(g, carry):
            idx16 = idx_v[pl.ds(g * 16, 16)]
            src = idx16
            dst = (g * 16 + lanes) * _DIM
            for _ in range(_DIM):
                plsc.store_scatter(rows_v, [dst], plsc.load_gather(table_v, [src]))
                src = src + _N_EMBED
                dst = dst + 1
            return carry

        lax.fori_loop(0, groups, body, 0)
        pltpu.sync_copy(rows_v, out_hbm.at[wid])

    return gather


def kernel(input, embed):
    b, t, c = input.shape
    flatten = input.reshape(-1, _DIM)
    rows = flatten.shape[0]
    ind, loss_sum = _dist_argmin(flatten, embed)
    table = embed.reshape(-1)
    quantize = _make_gather(rows)(table, ind)
    loss = loss_sum * (_BETA / (rows * _DIM))
    return quantize.reshape(b, t, c), loss, ind.reshape(b, t)
